# baseline (speedup 1.0000x reference)
import functools

import jax
import jax.numpy as jnp
from jax import lax
from jax.experimental import pallas as pl
from jax.experimental.pallas import tpu as pltpu
from jax.experimental.pallas import tpu_sc as plsc

BATCH = 4096
FAST_DIM = 64
VOCAB = 100000

_NC = 2
_NS = 16
_NW = _NC * _NS
_B_PER_W = BATCH // _NW

_BN = 512
_NFULL = VOCAB // _BN
_TAIL = VOCAB - _NFULL * _BN
_NBUF = 4


def _sc_gather_body(idx_hbm, table_hbm, out_hbm, idx_v, rows_v, sem):
    wid = lax.axis_index("s") * _NC + lax.axis_index("c")
    base = wid * _B_PER_W
    pltpu.sync_copy(idx_hbm.at[pl.ds(base, _B_PER_W)], idx_v)
    pltpu.async_copy(table_hbm.at[idx_v], rows_v, sem).wait()
    pltpu.sync_copy(rows_v, out_hbm.at[pl.ds(base, _B_PER_W)])


def _sc_gather(last_tok, emb_table):
    mesh = plsc.VectorSubcoreMesh(core_axis_name="c", subcore_axis_name="s")
    return pl.kernel(
        _sc_gather_body,
        mesh=mesh,
        out_type=jax.ShapeDtypeStruct((BATCH, FAST_DIM), jnp.float32),
        scratch_types=[
            pltpu.VMEM((_B_PER_W,), jnp.int32),
            pltpu.VMEM((_B_PER_W, FAST_DIM), jnp.float32),
            pltpu.SemaphoreType.DMA,
        ],
        compiler_params=pltpu.CompilerParams(use_tc_tiling_on_sc=False),
    )(last_tok, emb_table)


def _full_copy(acc, out_hbm, sems, slot, step):
    return pltpu.make_async_copy(
        acc.at[slot],
        out_hbm.at[:, pl.ds(step * _BN, _BN)],
        sems.at[slot],
    )


def _mm_body(emb_ref, w_ref, b_ref, out_hbm, acc, sems):
    i = pl.program_id(0)
    slot = lax.rem(i, _NBUF)

    @pl.when(i >= _NBUF)
    def _():
        _full_copy(acc, out_hbm, sems, slot, i - _NBUF).wait()

    acc[slot] = (
        jnp.dot(emb_ref[...], w_ref[...], preferred_element_type=jnp.float32)
        + b_ref[...]
    )

    _full_copy(acc, out_hbm, sems, slot, i).start()

    @pl.when(i == _NFULL - 1)
    def _():
        for k in range(_NBUF):
            s = _NFULL - _NBUF + k
            _full_copy(acc, out_hbm, sems, s % _NBUF, s).wait()


def _tc_project(fast_embed, W, b2d):
    return pl.pallas_call(
        _mm_body,
        grid=(_NFULL,),
        in_specs=[
            pl.BlockSpec((BATCH, FAST_DIM), lambda i: (0, 0)),
            pl.BlockSpec((FAST_DIM, _BN), lambda i: (0, i)),
            pl.BlockSpec((1, _BN), lambda i: (0, i)),
        ],
        out_specs=pl.BlockSpec(memory_space=pltpu.HBM),
        out_shape=jax.ShapeDtypeStruct((BATCH, VOCAB), jnp.float32),
        scratch_shapes=[
            pltpu.VMEM((_NBUF, BATCH, _BN), jnp.float32),
            pltpu.SemaphoreType.DMA((_NBUF,)),
        ],
        compiler_params=pltpu.CompilerParams(
            dimension_semantics=("arbitrary",),
        ),
    )(fast_embed, W, b2d)


def _tail_body(logits_ref, emb_ref, w_ref, b_ref, out_ref):
    del logits_ref
    out_ref[...] = (
        jnp.dot(emb_ref[...], w_ref[...], preferred_element_type=jnp.float32)
        + b_ref[...]
    )


def _tc_tail(logits, fast_embed, W, b2d):
    blk = 256
    last = VOCAB // blk
    return pl.pallas_call(
        _tail_body,
        grid=(1,),
        in_specs=[
            pl.BlockSpec(memory_space=pltpu.HBM),
            pl.BlockSpec((BATCH, FAST_DIM), lambda i: (0, 0)),
            pl.BlockSpec((FAST_DIM, blk), lambda i: (0, last)),
            pl.BlockSpec((1, blk), lambda i: (0, last)),
        ],
        out_specs=pl.BlockSpec((BATCH, blk), lambda i: (0, last)),
        out_shape=jax.ShapeDtypeStruct((BATCH, VOCAB), jnp.float32),
        input_output_aliases={0: 0},
        compiler_params=pltpu.CompilerParams(
            dimension_semantics=("arbitrary",),
        ),
    )(logits, fast_embed, W, b2d)


def kernel(x, emb_table, W, b):
    last_tok = x[:, -1].astype(jnp.int32)
    fast_embed = _sc_gather(last_tok, emb_table)
    b2d = b.reshape(1, VOCAB)
    logits = _tc_project(fast_embed, W, b2d)
    return _tc_tail(logits, fast_embed, W, b2d)
